# Initial kernel scaffold; baseline (speedup 1.0000x reference)
import functools

import jax
import jax.numpy as jnp
from jax import lax
from jax.experimental import pallas as pl
from jax.experimental.pallas import tpu as pltpu
from jax.experimental.pallas import tpu_sc as plsc

NC = 2
NS = 16
NW = NC * NS

D = 32
B = 16384 * 50
B_PER_W = B // NW
GROUP = 128
GROUPS_PER_CHUNK = 8
CHUNK = GROUP * GROUPS_PER_CHUNK
N_CHUNKS = B_PER_W // CHUNK

_mesh = plsc.VectorSubcoreMesh(
    core_axis_name="c", subcore_axis_name="s", num_cores=NC, num_subcores=NS
)


@functools.partial(
    pl.kernel,
    out_type=jax.ShapeDtypeStruct((B, D), jnp.float32),
    mesh=_mesh,
    scratch_types=[
        pltpu.VMEM((GROUPS_PER_CHUNK, GROUP), jnp.int32),
        pltpu.VMEM((CHUNK, D), jnp.float32),
        pltpu.SemaphoreType.DMA,
    ],
)
def _gather_kernel(idx_hbm, table_hbm, out_hbm, idx_v, rows_v, sem):
    wid = lax.axis_index("s") * NC + lax.axis_index("c")
    base = wid * B_PER_W

    @pl.loop(0, N_CHUNKS)
    def _chunk(g):
        off = base + g * CHUNK
        pltpu.sync_copy(idx_hbm.at[pl.ds(off, CHUNK)], idx_v)
        copies = []
        for j in range(GROUPS_PER_CHUNK):
            copies.append(
                pltpu.async_copy(
                    table_hbm.at[idx_v.at[j]],
                    rows_v.at[pl.ds(j * GROUP, GROUP)],
                    sem,
                )
            )
        for c in copies:
            c.wait()
        pltpu.sync_copy(rows_v, out_hbm.at[pl.ds(off, CHUNK)])


def kernel(x, table):
    flat = x.reshape(-1)
    out = _gather_kernel(flat, table)
    return out.reshape(x.shape + (D,))


# speedup vs baseline: 1.0946x; 1.0946x over previous
import functools

import jax
import jax.numpy as jnp
from jax import lax
from jax.experimental import pallas as pl
from jax.experimental.pallas import tpu as pltpu
from jax.experimental.pallas import tpu_sc as plsc

NC = 2
NS = 16
NW = NC * NS

D = 32
B = 16384 * 50
B_PER_W = B // NW
GROUP = 128
GROUPS_PER_CHUNK = 8
CHUNK = GROUP * GROUPS_PER_CHUNK
N_CHUNKS = B_PER_W // CHUNK

_mesh = plsc.VectorSubcoreMesh(
    core_axis_name="c", subcore_axis_name="s", num_cores=NC, num_subcores=NS
)


@functools.partial(
    pl.kernel,
    out_type=jax.ShapeDtypeStruct((B, D), jnp.float32),
    mesh=_mesh,
    scratch_types=[
        pltpu.VMEM((CHUNK,), jnp.int32),
        pltpu.VMEM((CHUNK, D), jnp.float32),
        pltpu.SemaphoreType.DMA,
    ],
    compiler_params=pltpu.CompilerParams(use_tc_tiling_on_sc=False),
)
def _gather_kernel(idx_hbm, table_hbm, out_hbm, idx_v, rows_v, sem):
    wid = lax.axis_index("s") * NC + lax.axis_index("c")
    base = wid * B_PER_W

    @pl.loop(0, N_CHUNKS)
    def _chunk(g):
        off = base + g * CHUNK
        pltpu.sync_copy(idx_hbm.at[pl.ds(off, CHUNK)], idx_v)
        copies = []
        for j in range(GROUPS_PER_CHUNK):
            copies.append(
                pltpu.async_copy(
                    table_hbm.at[idx_v.at[pl.ds(j * GROUP, GROUP)]],
                    rows_v.at[pl.ds(j * GROUP, GROUP)],
                    sem,
                )
            )
        for c in copies:
            c.wait()
        pltpu.sync_copy(rows_v, out_hbm.at[pl.ds(off, CHUNK)])


def kernel(x, table):
    flat = x.reshape(-1)
    out = _gather_kernel(flat, table)
    return out.reshape(x.shape + (D,))


# speedup vs baseline: 1.1133x; 1.0170x over previous
import functools

import jax
import jax.numpy as jnp
from jax import lax
from jax.experimental import pallas as pl
from jax.experimental.pallas import tpu as pltpu
from jax.experimental.pallas import tpu_sc as plsc

NC = 2
NS = 16
NW = NC * NS

D = 32
B = 16384 * 50
B_PER_W = B // NW
GROUP = 128
GROUPS_PER_CHUNK = 10
CHUNK = GROUP * GROUPS_PER_CHUNK
N_CHUNKS = B_PER_W // CHUNK

_mesh = plsc.VectorSubcoreMesh(
    core_axis_name="c", subcore_axis_name="s", num_cores=NC, num_subcores=NS
)


@functools.partial(
    pl.kernel,
    out_type=jax.ShapeDtypeStruct((B, D), jnp.float32),
    mesh=_mesh,
    scratch_types=[
        pltpu.VMEM((B_PER_W,), jnp.int32),
        pltpu.VMEM((CHUNK, D), jnp.float32),
        pltpu.VMEM((CHUNK, D), jnp.float32),
        pltpu.SemaphoreType.DMA,
        pltpu.SemaphoreType.DMA,
        pltpu.SemaphoreType.DMA,
        pltpu.SemaphoreType.DMA,
    ],
    compiler_params=pltpu.CompilerParams(use_tc_tiling_on_sc=False),
)
def _gather_kernel(
    idx_hbm, table_hbm, out_hbm, idx_v, rows0, rows1, g0, g1, o0, o1
):
    wid = lax.axis_index("s") * NC + lax.axis_index("c")
    base = wid * B_PER_W

    pltpu.sync_copy(idx_hbm.at[pl.ds(base, B_PER_W)], idx_v)

    def fire(c, rows, sem):
        for j in range(GROUPS_PER_CHUNK):
            pltpu.async_copy(
                table_hbm.at[idx_v.at[pl.ds(c * CHUNK + j * GROUP, GROUP)]],
                rows.at[pl.ds(j * GROUP, GROUP)],
                sem,
            )

    def wait_gathers(rows, sem):
        for j in range(GROUPS_PER_CHUNK):
            pltpu.make_async_copy(
                table_hbm.at[idx_v.at[pl.ds(j * GROUP, GROUP)]],
                rows.at[pl.ds(j * GROUP, GROUP)],
                sem,
            ).wait()

    def writeback(c, rows, sem):
        pltpu.async_copy(rows, out_hbm.at[pl.ds(base + c * CHUNK, CHUNK)], sem)

    def wait_writeback(rows, sem):
        pltpu.make_async_copy(
            rows, out_hbm.at[pl.ds(base, CHUNK)], sem
        ).wait()

    fire(0, rows0, g0)

    @pl.loop(0, N_CHUNKS, step=2)
    def _pipeline(g):
        @pl.when(g > 0)
        def _():
            wait_writeback(rows1, o1)

        fire(g + 1, rows1, g1)
        wait_gathers(rows0, g0)
        writeback(g, rows0, o0)

        @pl.when(g + 2 < N_CHUNKS)
        def _():
            wait_writeback(rows0, o0)
            fire(g + 2, rows0, g0)

        wait_gathers(rows1, g1)
        writeback(g + 1, rows1, o1)

    wait_writeback(rows0, o0)
    wait_writeback(rows1, o1)


def kernel(x, table):
    flat = x.reshape(-1)
    out = _gather_kernel(flat, table)
    return out.reshape(x.shape + (D,))


# speedup vs baseline: 1.7980x; 1.6151x over previous
import functools

import jax
import jax.numpy as jnp
from jax import lax
from jax.experimental import pallas as pl
from jax.experimental.pallas import tpu as pltpu
from jax.experimental.pallas import tpu_sc as plsc

NC = 2
NS = 16
NW = NC * NS

R = 16384
S = 50
D = 32
R_PER_W = R // NW
CHUNK = 16
N_CHUNKS = R_PER_W // CHUNK

_mesh = plsc.VectorSubcoreMesh(
    core_axis_name="c", subcore_axis_name="s", num_cores=NC, num_subcores=NS
)


@functools.partial(
    pl.kernel,
    out_type=jax.ShapeDtypeStruct((R, S, D), jnp.float32),
    mesh=_mesh,
    scratch_types=[
        pltpu.VMEM((CHUNK, S), jnp.int32),
        pltpu.VMEM((CHUNK, S), jnp.int32),
        pltpu.VMEM((CHUNK, S, D), jnp.float32),
        pltpu.VMEM((CHUNK, S, D), jnp.float32),
        pltpu.SemaphoreType.DMA,
        pltpu.SemaphoreType.DMA,
        pltpu.SemaphoreType.DMA,
        pltpu.SemaphoreType.DMA,
        pltpu.SemaphoreType.DMA,
        pltpu.SemaphoreType.DMA,
    ],
    compiler_params=pltpu.CompilerParams(use_tc_tiling_on_sc=False),
)
def _gather_kernel(
    x_hbm, table_hbm, out_hbm,
    idx0, idx1, rows0, rows1, i0, i1, g0, g1, o0, o1
):
    wid = lax.axis_index("s") * NC + lax.axis_index("c")
    base = wid * R_PER_W

    def stage_idx(c, idx, sem):
        pltpu.async_copy(x_hbm.at[pl.ds(base + c * CHUNK, CHUNK)], idx, sem)

    def wait_idx(idx, sem):
        pltpu.make_async_copy(x_hbm.at[pl.ds(base, CHUNK)], idx, sem).wait()

    def fire(idx, rows, sem):
        for r in range(CHUNK):
            pltpu.async_copy(table_hbm.at[idx.at[r]], rows.at[r], sem)

    def wait_gathers(idx, rows, sem):
        for r in range(CHUNK):
            pltpu.make_async_copy(
                table_hbm.at[idx.at[r]], rows.at[r], sem
            ).wait()

    def writeback(c, rows, sem):
        pltpu.async_copy(rows, out_hbm.at[pl.ds(base + c * CHUNK, CHUNK)], sem)

    def wait_writeback(rows, sem):
        pltpu.make_async_copy(
            rows, out_hbm.at[pl.ds(base, CHUNK)], sem
        ).wait()

    stage_idx(0, idx0, i0)
    stage_idx(1, idx1, i1)
    wait_idx(idx0, i0)
    fire(idx0, rows0, g0)

    @pl.loop(0, N_CHUNKS, step=2)
    def _pipeline(g):
        @pl.when(g > 0)
        def _():
            wait_writeback(rows1, o1)

        wait_idx(idx1, i1)
        fire(idx1, rows1, g1)
        wait_gathers(idx0, rows0, g0)
        writeback(g, rows0, o0)

        @pl.when(g + 2 < N_CHUNKS)
        def _():
            wait_writeback(rows0, o0)
            stage_idx(g + 2, idx0, i0)
            wait_idx(idx0, i0)
            fire(idx0, rows0, g0)

        wait_gathers(idx1, rows1, g1)
        writeback(g + 1, rows1, o1)

        @pl.when(g + 3 < N_CHUNKS)
        def _():
            stage_idx(g + 3, idx1, i1)

    wait_writeback(rows0, o0)
    wait_writeback(rows1, o1)


def kernel(x, table):
    return _gather_kernel(x, table)
